# Initial kernel scaffold; baseline (speedup 1.0000x reference)
import jax
import jax.numpy as jnp
from jax.experimental import pallas as pl


def kernel(hidden_states, w_gate, experts_gate, experts_up, experts_down, shared_gate, shared_up, shared_down):
    raise NotImplementedError("write your pallas kernel here")


# speedup vs baseline: 6.8914x; 6.8914x over previous
import functools

import jax
import jax.numpy as jnp
from jax import lax
from jax.experimental import pallas as pl
from jax.experimental.pallas import tpu as pltpu
from jax.experimental.pallas import tpu_sc as plsc

N = 2048
D = 1024
F = 512
E = 64
FS = 512
K = 2
TM = 128
NT = (N * K) // TM + E
NP = NT * TM


def _router_dispatch_body(x_ref, wg_ref, w_ref, pos_ref, texp_ref, tvalid_ref,
                          oh_ref):
    x = x_ref[...]
    logits = jnp.dot(x, wg_ref[...], preferred_element_type=jnp.float32)
    m = jnp.max(logits, axis=1, keepdims=True)
    p = jnp.exp(logits - m)
    p = p / jnp.sum(p, axis=1, keepdims=True)
    eidx = lax.broadcasted_iota(jnp.int32, (N, E), 1)
    m1 = jnp.max(p, axis=1, keepdims=True)
    i1 = jnp.min(jnp.where(p == m1, eidx, E), axis=1, keepdims=True)
    p2 = jnp.where(eidx == i1, -1.0, p)
    m2 = jnp.max(p2, axis=1, keepdims=True)
    i2 = jnp.min(jnp.where(p2 == m2, eidx, E), axis=1, keepdims=True)
    s = m1 + m2
    w_ref[...] = jnp.concatenate([m1 / s, m2 / s], axis=1)

    e_all = jnp.concatenate([i1, i2], axis=0)
    oh = (e_all == lax.broadcasted_iota(jnp.int32, (K * N, E), 1))
    oh = oh.astype(jnp.float32)
    oh_ref[...] = oh
    counts = jnp.sum(oh, axis=0, keepdims=True)
    n_tiles = jnp.floor((counts + (TM - 1.0)) * (1.0 / TM))
    triu = (lax.broadcasted_iota(jnp.int32, (E, E), 0)
            < lax.broadcasted_iota(jnp.int32, (E, E), 1)).astype(jnp.float32)
    tile_start = jnp.dot(n_tiles, triu, preferred_element_type=jnp.float32)
    addend = tile_start * float(TM) - 1.0

    tri = (lax.broadcasted_iota(jnp.int32, (TM, TM), 0)
           >= lax.broadcasted_iota(jnp.int32, (TM, TM), 1)).astype(jnp.float32)

    def body(b, prefix):
        blk = oh_ref[pl.ds(b * TM, TM), :]
        c = jnp.dot(tri, blk, preferred_element_type=jnp.float32) + prefix
        posb = jnp.sum(blk * (c + addend), axis=1, keepdims=True)
        pos_ref[pl.ds(b * TM, TM), :] = posb.astype(jnp.int32)
        return prefix + jnp.sum(blk, axis=0, keepdims=True)

    lax.fori_loop(0, (K * N) // TM, body, jnp.zeros((1, E), jnp.float32))

    t_iota = lax.broadcasted_iota(jnp.int32, (TM, E), 0).astype(jnp.float32)
    texp = jnp.sum((tile_start <= t_iota).astype(jnp.float32),
                   axis=1, keepdims=True) - 1.0
    texp_ref[...] = jnp.maximum(texp, 0.0).astype(jnp.int32)
    total = jnp.sum(n_tiles, axis=1, keepdims=True)
    tvalid_ref[...] = (t_iota[:, 0:1] < total).astype(jnp.int32)


def _router_dispatch(x, w_gate):
    return pl.pallas_call(
        _router_dispatch_body,
        out_shape=(
            jax.ShapeDtypeStruct((N, K), jnp.float32),
            jax.ShapeDtypeStruct((K * N, 1), jnp.int32),
            jax.ShapeDtypeStruct((TM, 1), jnp.int32),
            jax.ShapeDtypeStruct((TM, 1), jnp.int32),
        ),
        scratch_shapes=[pltpu.VMEM((K * N, E), jnp.float32)],
    )(x, w_gate)


def _make_sc_mesh():
    return plsc.VectorSubcoreMesh(core_axis_name="c", subcore_axis_name="s")


def _sc_scatter(x, pos):
    mesh = _make_sc_mesh()
    nw = mesh.num_cores * mesh.num_subcores
    chunk = N // nw

    @functools.partial(
        pl.kernel,
        out_type=jax.ShapeDtypeStruct((NP, D), jnp.float32),
        mesh=mesh,
        scratch_types=[
            pltpu.VMEM((chunk,), jnp.int32),
            pltpu.VMEM((chunk,), jnp.int32),
            pltpu.VMEM((chunk, D), jnp.float32),
            pltpu.SemaphoreType.DMA,
        ],
    )
    def body(x_hbm, pos_hbm, out_hbm, idx0_v, idx1_v, rows_v, sem):
        wid = lax.axis_index("s") * mesh.num_cores + lax.axis_index("c")
        base = wid * chunk
        pltpu.sync_copy(x_hbm.at[pl.ds(base, chunk)], rows_v)
        pltpu.sync_copy(pos_hbm.at[pl.ds(base, chunk)], idx0_v)
        pltpu.sync_copy(pos_hbm.at[pl.ds(N + base, chunk)], idx1_v)
        c0 = pltpu.async_copy(rows_v, out_hbm.at[idx0_v], sem)
        c1 = pltpu.async_copy(rows_v, out_hbm.at[idx1_v], sem)
        c0.wait()
        c1.wait()

    return body(x, pos)


def _sc_gather(y, pos):
    mesh = _make_sc_mesh()
    nw = mesh.num_cores * mesh.num_subcores
    chunk = N // nw

    @functools.partial(
        pl.kernel,
        out_type=(jax.ShapeDtypeStruct((N, D), jnp.float32),
                  jax.ShapeDtypeStruct((N, D), jnp.float32)),
        mesh=mesh,
        scratch_types=[
            pltpu.VMEM((chunk,), jnp.int32),
            pltpu.VMEM((chunk, D), jnp.float32),
            pltpu.SemaphoreType.DMA,
        ],
    )
    def body(y_hbm, pos_hbm, g0_hbm, g1_hbm, idx_v, rows_v, sem):
        wid = lax.axis_index("s") * mesh.num_cores + lax.axis_index("c")
        base = wid * chunk
        pltpu.sync_copy(pos_hbm.at[pl.ds(base, chunk)], idx_v)
        pltpu.async_copy(y_hbm.at[idx_v], rows_v, sem).wait()
        pltpu.sync_copy(rows_v, g0_hbm.at[pl.ds(base, chunk)])
        pltpu.sync_copy(pos_hbm.at[pl.ds(N + base, chunk)], idx_v)
        pltpu.async_copy(y_hbm.at[idx_v], rows_v, sem).wait()
        pltpu.sync_copy(rows_v, g1_hbm.at[pl.ds(base, chunk)])

    return body(y, pos)


def _ffn_body(texp_ref, tvalid_ref, xs_ref, wg_ref, wu_ref, wd_ref, y_ref):
    t = pl.program_id(0)

    @pl.when(tvalid_ref[t] > 0)
    def _():
        xb = xs_ref[...]
        g = jnp.dot(xb, wg_ref[0], preferred_element_type=jnp.float32)
        u = jnp.dot(xb, wu_ref[0], preferred_element_type=jnp.float32)
        h = (g * jax.nn.sigmoid(g)) * u
        y_ref[...] = jnp.dot(h, wd_ref[0], preferred_element_type=jnp.float32)


def _grouped_ffn(texp, tvalid, x_sorted, eg, eu, ed):
    grid_spec = pltpu.PrefetchScalarGridSpec(
        num_scalar_prefetch=2,
        grid=(NT,),
        in_specs=[
            pl.BlockSpec((TM, D), lambda t, texp, tvalid: (t, 0)),
            pl.BlockSpec((1, D, F), lambda t, texp, tvalid: (texp[t], 0, 0)),
            pl.BlockSpec((1, D, F), lambda t, texp, tvalid: (texp[t], 0, 0)),
            pl.BlockSpec((1, F, D), lambda t, texp, tvalid: (texp[t], 0, 0)),
        ],
        out_specs=pl.BlockSpec((TM, D), lambda t, texp, tvalid: (t, 0)),
    )
    return pl.pallas_call(
        _ffn_body,
        grid_spec=grid_spec,
        out_shape=jax.ShapeDtypeStruct((NP, D), jnp.float32),
    )(texp, tvalid, x_sorted, eg, eu, ed)


def _combine_body(g0_ref, g1_ref, w_ref, x_ref, sg_ref, su_ref, sd_ref, o_ref):
    w = w_ref[...]
    xb = x_ref[...]
    a = jnp.dot(xb, sg_ref[...], preferred_element_type=jnp.float32)
    b = jnp.dot(xb, su_ref[...], preferred_element_type=jnp.float32)
    sh = (a * jax.nn.sigmoid(a)) * b
    o_ref[...] = (w[:, 0:1] * g0_ref[...] + w[:, 1:2] * g1_ref[...]
                  + jnp.dot(sh, sd_ref[...], preferred_element_type=jnp.float32))


def _combine(g0, g1, topk_w, x, sg, su, sd):
    tb = 256
    grid = (N // tb,)
    return pl.pallas_call(
        _combine_body,
        grid=grid,
        in_specs=[
            pl.BlockSpec((tb, D), lambda i: (i, 0)),
            pl.BlockSpec((tb, D), lambda i: (i, 0)),
            pl.BlockSpec((tb, K), lambda i: (i, 0)),
            pl.BlockSpec((tb, D), lambda i: (i, 0)),
            pl.BlockSpec((D, FS), lambda i: (0, 0)),
            pl.BlockSpec((D, FS), lambda i: (0, 0)),
            pl.BlockSpec((FS, D), lambda i: (0, 0)),
        ],
        out_specs=pl.BlockSpec((tb, D), lambda i: (i, 0)),
        out_shape=jax.ShapeDtypeStruct((N, D), jnp.float32),
    )(g0, g1, topk_w, x, sg, su, sd)


def kernel(hidden_states, w_gate, experts_gate, experts_up, experts_down,
           shared_gate, shared_up, shared_down):
    x = hidden_states
    topk_w, pos, texp, tvalid = _router_dispatch(x, w_gate)
    pos_flat = pos.reshape(K * N)
    texp_flat = texp.reshape(TM)[:NT]
    tvalid_flat = tvalid.reshape(TM)[:NT]
    x_sorted = _sc_scatter(x, pos_flat)
    y = _grouped_ffn(texp_flat, tvalid_flat, x_sorted,
                     experts_gate, experts_up, experts_down)
    g0, g1 = _sc_gather(y, pos_flat)
    return _combine(g0, g1, topk_w, x, shared_gate, shared_up, shared_down)
